# baseline (speedup 1.0000x reference)
import jax
import jax.numpy as jnp
from jax.experimental import pallas as pl
from jax.experimental.pallas import tpu as pltpu

_N, _F, _H = 10000, 128, 128
_BM = 200
_K = 4
_STEPS = _N // _BM


def _gcn_body(embs_ref, w_ref, adj_ref, b_ref, a_ref, out_ref, bufs, sf_ref, sems):
    def copy_in(step, slot):
        return pltpu.make_async_copy(
            adj_ref.at[pl.ds(step * _BM, _BM), :], bufs.at[slot], sems.at[slot]
        )

    for j in range(_K):
        copy_in(j, j).start()

    sf_ref[...] = jnp.dot(
        embs_ref[...].astype(jnp.bfloat16),
        w_ref[...].astype(jnp.bfloat16),
        preferred_element_type=jnp.float32,
    ).astype(jnp.bfloat16)

    bias = b_ref[...]
    a = a_ref[0, 0]

    def step_fn(i, carry):
        slot = jax.lax.rem(i, _K)
        copy_in(i, slot).wait()
        acc = jnp.dot(
            bufs[slot].astype(jnp.bfloat16),
            sf_ref[...],
            preferred_element_type=jnp.float32,
        )
        o = acc + bias
        out_ref[pl.ds(i * _BM, _BM), :] = jnp.where(o > 0, o, a * o)

        nxt = i + _K

        @pl.when(nxt < _STEPS)
        def _():
            copy_in(nxt, slot).start()

        return carry

    jax.lax.fori_loop(0, _STEPS, step_fn, 0)


def kernel(embs, adj, W, b, prelu_a):
    return pl.pallas_call(
        _gcn_body,
        in_specs=[
            pl.BlockSpec(memory_space=pltpu.VMEM),
            pl.BlockSpec(memory_space=pltpu.VMEM),
            pl.BlockSpec(memory_space=pl.ANY),
            pl.BlockSpec(memory_space=pltpu.VMEM),
            pl.BlockSpec(memory_space=pltpu.SMEM),
        ],
        out_specs=pl.BlockSpec(memory_space=pltpu.VMEM),
        out_shape=jax.ShapeDtypeStruct((_N, _H), jnp.float32),
        scratch_shapes=[
            pltpu.VMEM((_K, _BM, _N), jnp.float32),
            pltpu.VMEM((_N, _H), jnp.bfloat16),
            pltpu.SemaphoreType.DMA((_K,)),
        ],
    )(embs, W, adj, b.reshape(1, _H), prelu_a.reshape(1, 1))


# speedup vs baseline: 1.0209x; 1.0209x over previous
import jax
import jax.numpy as jnp
from jax.experimental import pallas as pl
from jax.experimental.pallas import tpu as pltpu

_N, _F, _H = 10000, 128, 128
_BM = 400


def _gcn_body(embs_ref, w_ref, adj_ref, b_ref, a_ref, out_ref, sf_ref):
    m = pl.program_id(0)

    @pl.when(m == 0)
    def _():
        sf_ref[...] = jnp.dot(
            embs_ref[...].astype(jnp.bfloat16),
            w_ref[...].astype(jnp.bfloat16),
            preferred_element_type=jnp.float32,
        ).astype(jnp.bfloat16)

    acc = jnp.dot(
        adj_ref[...].astype(jnp.bfloat16),
        sf_ref[...],
        preferred_element_type=jnp.float32,
    )
    o = acc + b_ref[...]
    a = a_ref[0, 0]
    out_ref[...] = jnp.where(o > 0, o, a * o)


def kernel(embs, adj, W, b, prelu_a):
    return pl.pallas_call(
        _gcn_body,
        grid=(_N // _BM,),
        in_specs=[
            pl.BlockSpec((_N, _F), lambda m: (0, 0)),
            pl.BlockSpec((_F, _H), lambda m: (0, 0)),
            pl.BlockSpec((_BM, _N), lambda m: (m, 0)),
            pl.BlockSpec((1, _H), lambda m: (0, 0)),
            pl.BlockSpec((1, 1), lambda m: (0, 0), memory_space=pltpu.SMEM),
        ],
        out_specs=pl.BlockSpec((_BM, _H), lambda m: (m, 0)),
        out_shape=jax.ShapeDtypeStruct((_N, _H), jnp.float32),
        scratch_shapes=[pltpu.VMEM((_N, _H), jnp.bfloat16)],
    )(embs, W, adj, b.reshape(1, _H), prelu_a.reshape(1, 1))
